# baseline (speedup 1.0000x reference)
import functools

import jax
import jax.numpy as jnp
from jax import lax
from jax.experimental import pallas as pl
from jax.experimental.pallas import tpu as pltpu
from jax.experimental.pallas import tpu_sc as plsc

B = 16384
F_CAT = 26
VOCAB = 100000
F_NUM = 13
D = 32
F_TOT = F_CAT + F_NUM

NW = 32
C = 128
SK = 4

CAT_TOT = B * F_CAT
NUM_TOT = B * F_NUM
CAT_PW = CAT_TOT // NW
NUM_PW = NUM_TOT // NW
CAT_CH = CAT_PW // C
NUM_CH = NUM_PW // C
NSK = CAT_CH // SK
ROWS = B * F_TOT

EPS = 1e-5

_mesh = plsc.VectorSubcoreMesh(core_axis_name="c", subcore_axis_name="s")


def _hsum(x):
    s = x[0]
    for i in range(1, 16):
        s = s + x[i]
    return s


@functools.partial(
    pl.kernel,
    mesh=_mesh,
    out_type=jax.ShapeDtypeStruct((ROWS, D), jnp.float32),
    compiler_params=pltpu.CompilerParams(
        needs_layout_passes=False, use_tc_tiling_on_sc=False),
    scratch_types=[
        pltpu.VMEM((SK, C), jnp.int32),
        pltpu.VMEM((SK, C), jnp.int32),
        pltpu.VMEM((SK, C), jnp.int32),
        pltpu.VMEM((SK * C, 128), jnp.float32),
        pltpu.VMEM((SK * C, D), jnp.float32),
        pltpu.VMEM((NUM_PW,), jnp.float32),
        pltpu.VMEM((2, C), jnp.int32),
        pltpu.VMEM((C, D), jnp.float32),
        pltpu.VMEM((C, D), jnp.float32),
        pltpu.VMEM((128,), jnp.float32),
        pltpu.SemaphoreType.DMA,
        pltpu.SemaphoreType.DMA,
    ],
)
def _sc_embed(tflat, cat4, nv2, prm, out, cidx, qidx, didx, qrows, rows,
              nv_all, ndst, nra, nrb, pp, gsem, ssem):
    wid = lax.axis_index("s") * 2 + lax.axis_index("c")
    ii = lax.iota(jnp.int32, 16)
    i39 = ii * F_TOT

    def select_chunk(j, carry):
        def grp_body(g, carry2):
            gbase = j * C + g * 16
            row16 = ii + gbase
            q32 = qidx[j, pl.ds(g * 16, 16)] * D
            z16 = ii * 0
            for d in range(D):
                col = q32 + d
                val = plsc.load_gather(qrows, [row16, col])
                plsc.store_scatter(rows, [row16, z16 + d], val)
            return carry2
        return lax.fori_loop(0, C // 16, grp_body, carry)

    def cat_super(k, carry):
        pltpu.sync_copy(cat4.at[wid, k], cidx)
        pos0 = wid * CAT_PW + k * (SK * C)
        f = pos0 // B
        voff = f * VOCAB
        for j in range(SK):
            bb = (pos0 + j * C) % B
            for grp in range(C // 16):
                sl = pl.ds(grp * 16, 16)
                full = cidx[j, sl] + voff
                cidx[j, sl] = full >> 2
                qidx[j, sl] = full & 3
                didx[j, sl] = i39 + ((bb + grp * 16) * F_TOT + f)
        gs = [
            pltpu.async_copy(
                tflat.at[cidx.at[j]], qrows.at[pl.ds(j * C, C)], gsem)
            for j in range(SK)
        ]
        for g in gs:
            g.wait()
        lax.fori_loop(0, SK, select_chunk, 0)
        ss = [
            pltpu.async_copy(
                rows.at[pl.ds(j * C, C)], out.at[didx.at[j]], ssem)
            for j in range(SK)
        ]
        for s in ss:
            s.wait()
        return carry

    lax.fori_loop(0, NSK, cat_super, 0)

    pltpu.sync_copy(nv2.at[wid], nv_all)
    pltpu.sync_copy(prm, pp)

    w0 = pp[pl.ds(0, 16)]
    w1 = pp[pl.ds(16, 16)]
    b0 = pp[pl.ds(32, 16)]
    b1 = pp[pl.ds(48, 16)]
    mw = (_hsum(w0) + _hsum(w1)) * (1.0 / D)
    mb = (_hsum(b0) + _hsum(b1)) * (1.0 / D)
    wc0 = w0 - mw
    wc1 = w1 - mw
    bc0 = b0 - mb
    bc1 = b1 - mb
    va = (_hsum(wc0 * wc0) + _hsum(wc1 * wc1)) * (1.0 / D)
    vc = (_hsum(wc0 * bc0) + _hsum(wc1 * bc1)) * (2.0 / D)
    vb = (_hsum(bc0 * bc0) + _hsum(bc1 * bc1)) * (1.0 / D) + EPS
    g0 = pp[pl.ds(64, 16)]
    g1 = pp[pl.ds(80, 16)]
    p0 = wc0 * g0
    p1 = wc1 * g1
    q0 = bc0 * g0
    q1 = bc1 * g1
    t0 = pp[pl.ds(96, 16)]
    t1 = pp[pl.ds(112, 16)]

    nbase = wid * NUM_PW

    def num_compute(chunk, slot, buf):
        pos0 = nbase + chunk * C
        jcol = pos0 // B
        bb = pos0 % B
        for grp in range(C // 16):
            sl = pl.ds(grp * 16, 16)
            ndst[slot, sl] = i39 + ((bb + grp * 16) * F_TOT + (F_CAT + jcol))
            v = nv_all[pl.ds(chunk * C + grp * 16, 16)]
            var = (v * v) * va + v * vc + vb
            z = jnp.maximum(var, 1.0)
            for _ in range(14):
                z = 0.5 * (z + var / z)
            y = 1.0 / z
            tvals = v * y
            for p in range(16):
                r = grp * 16 + p
                ts = tvals[p]
                us = y[p]
                buf[r, pl.ds(0, 16)] = ts * p0 + us * q0 + t0
                buf[r, pl.ds(16, 16)] = ts * p1 + us * q1 + t1

    def num_pair(m, carry):
        c0 = m * 2
        num_compute(c0, 0, nra)
        sa = pltpu.async_copy(nra, out.at[ndst.at[0]], ssem)
        num_compute(c0 + 1, 1, nrb)
        sa.wait()
        sb = pltpu.async_copy(nrb, out.at[ndst.at[1]], ssem)
        sb.wait()
        return carry

    lax.fori_loop(0, NUM_CH // 2, num_pair, 0)


def kernel(categorical_features, numerical_features, tables, W_num, b_num,
           ln_gamma, ln_beta):
    cat4 = categorical_features.astype(jnp.int32).T.reshape(NW, NSK, SK, C)
    nv2 = numerical_features.astype(jnp.float32).T.reshape(NW, NUM_PW)
    prm = jnp.concatenate([
        W_num.astype(jnp.float32), b_num.astype(jnp.float32),
        ln_gamma.astype(jnp.float32), ln_beta.astype(jnp.float32)])
    tflat = tables.reshape(F_CAT * VOCAB * D // 128, 128)
    out = _sc_embed(tflat, cat4, nv2, prm)
    return out.reshape(B, F_TOT, D)


# speedup vs baseline: 1.3352x; 1.3352x over previous
import functools

import jax
import jax.numpy as jnp
from jax import lax
from jax.experimental import pallas as pl
from jax.experimental.pallas import tpu as pltpu
from jax.experimental.pallas import tpu_sc as plsc

B = 16384
F_CAT = 26
VOCAB = 100000
F_NUM = 13
D = 32
F_TOT = F_CAT + F_NUM

NW = 32
C = 128
SK = 8

CAT_TOT = B * F_CAT
NUM_TOT = B * F_NUM
CAT_PW = CAT_TOT // NW
NUM_PW = NUM_TOT // NW
CAT_CH = CAT_PW // C
NUM_CH = NUM_PW // C
NSK = CAT_CH // SK
ROWS = B * F_TOT

EPS = 1e-5

_mesh = plsc.VectorSubcoreMesh(core_axis_name="c", subcore_axis_name="s")


def _hsum(x):
    s = x[0]
    for i in range(1, 16):
        s = s + x[i]
    return s


@functools.partial(
    pl.kernel,
    mesh=_mesh,
    out_type=jax.ShapeDtypeStruct((ROWS, D), jnp.float32),
    compiler_params=pltpu.CompilerParams(
        needs_layout_passes=False, use_tc_tiling_on_sc=False),
    scratch_types=[
        pltpu.VMEM((SK, C), jnp.int32),
        pltpu.VMEM((SK, C), jnp.int32),
        pltpu.VMEM((SK * C, D), jnp.float32),
        pltpu.VMEM((NUM_PW,), jnp.float32),
        pltpu.VMEM((2, C), jnp.int32),
        pltpu.VMEM((C, D), jnp.float32),
        pltpu.VMEM((C, D), jnp.float32),
        pltpu.VMEM((128,), jnp.float32),
        pltpu.SemaphoreType.DMA,
        pltpu.SemaphoreType.DMA,
    ],
)
def _sc_embed(tflat, cat4, nv2, prm, out, cidx, didx, rows, nv_all, ndst,
              nra, nrb, pp, gsem, ssem):
    wid = lax.axis_index("s") * 2 + lax.axis_index("c")
    i39 = lax.iota(jnp.int32, 16) * F_TOT

    def cat_super(k, carry):
        pltpu.sync_copy(cat4.at[wid, k], cidx)
        pos0 = wid * CAT_PW + k * (SK * C)
        f = pos0 // B
        voff = f * VOCAB
        for j in range(SK):
            bb = (pos0 + j * C) % B
            for grp in range(C // 16):
                sl = pl.ds(grp * 16, 16)
                cidx[j, sl] = cidx[j, sl] + voff
                didx[j, sl] = i39 + ((bb + grp * 16) * F_TOT + f)
        gs = [
            pltpu.async_copy(
                tflat.at[cidx.at[j]], rows.at[pl.ds(j * C, C)], gsem)
            for j in range(SK)
        ]
        for g in gs:
            g.wait()
        ss = [
            pltpu.async_copy(
                rows.at[pl.ds(j * C, C)], out.at[didx.at[j]], ssem)
            for j in range(SK)
        ]
        for s in ss:
            s.wait()
        return carry

    lax.fori_loop(0, NSK, cat_super, 0)

    pltpu.sync_copy(nv2.at[wid], nv_all)
    pltpu.sync_copy(prm, pp)

    w0 = pp[pl.ds(0, 16)]
    w1 = pp[pl.ds(16, 16)]
    b0 = pp[pl.ds(32, 16)]
    b1 = pp[pl.ds(48, 16)]
    mw = (_hsum(w0) + _hsum(w1)) * (1.0 / D)
    mb = (_hsum(b0) + _hsum(b1)) * (1.0 / D)
    wc0 = w0 - mw
    wc1 = w1 - mw
    bc0 = b0 - mb
    bc1 = b1 - mb
    va = (_hsum(wc0 * wc0) + _hsum(wc1 * wc1)) * (1.0 / D)
    vc = (_hsum(wc0 * bc0) + _hsum(wc1 * bc1)) * (2.0 / D)
    vb = (_hsum(bc0 * bc0) + _hsum(bc1 * bc1)) * (1.0 / D) + EPS
    g0 = pp[pl.ds(64, 16)]
    g1 = pp[pl.ds(80, 16)]
    p0 = wc0 * g0
    p1 = wc1 * g1
    q0 = bc0 * g0
    q1 = bc1 * g1
    t0 = pp[pl.ds(96, 16)]
    t1 = pp[pl.ds(112, 16)]

    nbase = wid * NUM_PW

    def num_compute(chunk, slot, buf):
        pos0 = nbase + chunk * C
        jcol = pos0 // B
        bb = pos0 % B
        for grp in range(C // 16):
            sl = pl.ds(grp * 16, 16)
            ndst[slot, sl] = i39 + ((bb + grp * 16) * F_TOT + (F_CAT + jcol))
            v = nv_all[pl.ds(chunk * C + grp * 16, 16)]
            var = (v * v) * va + v * vc + vb
            z = jnp.maximum(var, 1.0)
            for _ in range(14):
                z = 0.5 * (z + var / z)
            y = 1.0 / z
            tvals = v * y
            for p in range(16):
                r = grp * 16 + p
                ts = tvals[p]
                us = y[p]
                buf[r, pl.ds(0, 16)] = ts * p0 + us * q0 + t0
                buf[r, pl.ds(16, 16)] = ts * p1 + us * q1 + t1

    def num_pair(m, carry):
        c0 = m * 2
        num_compute(c0, 0, nra)
        sa = pltpu.async_copy(nra, out.at[ndst.at[0]], ssem)
        num_compute(c0 + 1, 1, nrb)
        sa.wait()
        sb = pltpu.async_copy(nrb, out.at[ndst.at[1]], ssem)
        sb.wait()
        return carry

    lax.fori_loop(0, NUM_CH // 2, num_pair, 0)


def kernel(categorical_features, numerical_features, tables, W_num, b_num,
           ln_gamma, ln_beta):
    cat4 = categorical_features.astype(jnp.int32).T.reshape(NW, NSK, SK, C)
    nv2 = numerical_features.astype(jnp.float32).T.reshape(NW, NUM_PW)
    prm = jnp.concatenate([
        W_num.astype(jnp.float32), b_num.astype(jnp.float32),
        ln_gamma.astype(jnp.float32), ln_beta.astype(jnp.float32)])
    tflat = tables.reshape(F_CAT * VOCAB, D)
    out = _sc_embed(tflat, cat4, nv2, prm)
    return out.reshape(B, F_TOT, D)
